# Initial kernel scaffold; baseline (speedup 1.0000x reference)
import jax
import jax.numpy as jnp
from jax.experimental import pallas as pl


def kernel(indices, W):
    raise NotImplementedError("write your pallas kernel here")


# speedup vs baseline: 14.2031x; 14.2031x over previous
import functools

import jax
import jax.numpy as jnp
from jax import lax
from jax.experimental import pallas as pl
from jax.experimental.pallas import tpu as pltpu
from jax.experimental.pallas import tpu_sc as plsc

VOCAB = 100000
EMBED_DIM = 64
BATCH = 16384
HIST = 50

NUM_CORES = 2
NUM_SUBCORES = 16
NUM_WORKERS = NUM_CORES * NUM_SUBCORES
BAGS_PER_WORKER = BATCH // NUM_WORKERS
CHUNK = 16
NUM_CHUNKS = BAGS_PER_WORKER // CHUNK
ROWS = CHUNK * HIST
NLANE = 16
DCHUNKS = EMBED_DIM // NLANE

_mesh = plsc.VectorSubcoreMesh(
    core_axis_name="c", subcore_axis_name="s",
    num_cores=NUM_CORES, num_subcores=NUM_SUBCORES)


@functools.partial(
    pl.kernel,
    mesh=_mesh,
    out_type=jax.ShapeDtypeStruct((BATCH, EMBED_DIM), jnp.float32),
    scratch_types=[
        pltpu.VMEM((ROWS,), jnp.int32),
        pltpu.VMEM((ROWS, EMBED_DIM), jnp.float32),
        pltpu.VMEM((CHUNK, EMBED_DIM), jnp.float32),
        pltpu.SemaphoreType.DMA,
    ],
    compiler_params=pltpu.CompilerParams(use_tc_tiling_on_sc=False),
)
def _embed_bag_mean(idx_hbm, w_hbm, out_hbm, idx_v, rows_v, out_v, sem):
    wid = lax.axis_index("s") * NUM_CORES + lax.axis_index("c")
    base_bag = wid * BAGS_PER_WORKER

    def chunk_body(ci, carry):
        bag0 = base_bag + ci * CHUNK
        pltpu.sync_copy(idx_hbm.at[pl.ds(bag0 * HIST, ROWS)], idx_v)
        pltpu.async_copy(w_hbm.at[idx_v], rows_v, sem).wait()

        def bag_body(bi, carry2):
            def row_body(ri, acc):
                r = bi * HIST + ri
                return tuple(
                    acc[c] + rows_v[r, pl.ds(c * NLANE, NLANE)]
                    for c in range(DCHUNKS)
                )

            acc = lax.fori_loop(
                0, HIST, row_body,
                tuple(jnp.zeros((NLANE,), jnp.float32) for _ in range(DCHUNKS)))
            for c in range(DCHUNKS):
                out_v[bi, pl.ds(c * NLANE, NLANE)] = jnp.maximum(
                    acc[c] * (1.0 / HIST), 0.0)
            return carry2

        lax.fori_loop(0, CHUNK, bag_body, 0)
        pltpu.sync_copy(out_v, out_hbm.at[pl.ds(bag0, CHUNK)])
        return carry

    lax.fori_loop(0, NUM_CHUNKS, chunk_body, 0)


def kernel(indices, W):
    idx_flat = indices.reshape(-1).astype(jnp.int32)
    return _embed_bag_mean(idx_flat, W)


# speedup vs baseline: 20.5386x; 1.4461x over previous
import functools

import jax
import jax.numpy as jnp
from jax import lax
from jax.experimental import pallas as pl
from jax.experimental.pallas import tpu as pltpu
from jax.experimental.pallas import tpu_sc as plsc

VOCAB = 100000
EMBED_DIM = 64
BATCH = 16384
HIST = 50

NUM_CORES = 2
NUM_SUBCORES = 16
NUM_WORKERS = NUM_CORES * NUM_SUBCORES
BAGS_PER_WORKER = BATCH // NUM_WORKERS
CHUNK = 16
NUM_CHUNKS = BAGS_PER_WORKER // CHUNK
ROWS = CHUNK * HIST
NLANE = 16
DCHUNKS = EMBED_DIM // NLANE

_mesh = plsc.VectorSubcoreMesh(
    core_axis_name="c", subcore_axis_name="s",
    num_cores=NUM_CORES, num_subcores=NUM_SUBCORES)


@functools.partial(
    pl.kernel,
    mesh=_mesh,
    out_type=jax.ShapeDtypeStruct((BATCH, EMBED_DIM), jnp.float32),
    scratch_types=[
        pltpu.VMEM((2, ROWS), jnp.int32),
        pltpu.VMEM((ROWS, EMBED_DIM), jnp.float32),
        pltpu.VMEM((ROWS, EMBED_DIM), jnp.float32),
        pltpu.VMEM((CHUNK, EMBED_DIM), jnp.float32),
        pltpu.SemaphoreType.DMA,
        pltpu.SemaphoreType.DMA,
    ],
    compiler_params=pltpu.CompilerParams(use_tc_tiling_on_sc=False),
)
def _embed_bag_mean(idx_hbm, w_hbm, out_hbm, idx_v, rows0_v, rows1_v, out_v,
                    sem0, sem1):
    wid = lax.axis_index("s") * NUM_CORES + lax.axis_index("c")
    base_bag = wid * BAGS_PER_WORKER
    rows_bufs = (rows0_v, rows1_v)
    sems = (sem0, sem1)

    pltpu.sync_copy(idx_hbm.at[pl.ds(base_bag * HIST, ROWS)], idx_v.at[0])
    pltpu.async_copy(w_hbm.at[idx_v.at[0]], rows_bufs[0], sems[0])

    def process(ci, rows_v):
        bag0 = base_bag + ci * CHUNK

        def bag_body(bi, carry2):
            acc = [rows_v[bi * HIST, pl.ds(c * NLANE, NLANE)]
                   for c in range(DCHUNKS)]
            for ri in range(1, HIST):
                for c in range(DCHUNKS):
                    acc[c] = acc[c] + rows_v[bi * HIST + ri,
                                             pl.ds(c * NLANE, NLANE)]
            for c in range(DCHUNKS):
                out_v[bi, pl.ds(c * NLANE, NLANE)] = jnp.maximum(
                    acc[c] * (1.0 / HIST), 0.0)
            return carry2

        lax.fori_loop(0, CHUNK, bag_body, 0)
        pltpu.sync_copy(out_v, out_hbm.at[pl.ds(bag0, CHUNK)])

    def outer(ci2, carry):
        for b in range(2):
            ci = ci2 * 2 + b
            nb = 1 - b

            @pl.when(ci + 1 < NUM_CHUNKS)
            def _():
                nbag0 = base_bag + (ci + 1) * CHUNK
                pltpu.sync_copy(idx_hbm.at[pl.ds(nbag0 * HIST, ROWS)],
                                idx_v.at[nb])
                pltpu.async_copy(w_hbm.at[idx_v.at[nb]], rows_bufs[nb],
                                 sems[nb])

            pltpu.make_async_copy(w_hbm.at[idx_v.at[b]], rows_bufs[b],
                                  sems[b]).wait()
            process(ci, rows_bufs[b])
        return carry

    lax.fori_loop(0, NUM_CHUNKS // 2, outer, 0)


def kernel(indices, W):
    idx_flat = indices.reshape(-1).astype(jnp.int32)
    return _embed_bag_mean(idx_flat, W)


# speedup vs baseline: 22.4255x; 1.0919x over previous
import functools

import jax
import jax.numpy as jnp
from jax import lax
from jax.experimental import pallas as pl
from jax.experimental.pallas import tpu as pltpu
from jax.experimental.pallas import tpu_sc as plsc

VOCAB = 100000
EMBED_DIM = 64
BATCH = 16384
HIST = 50

NUM_CORES = 2
NUM_SUBCORES = 16
NUM_WORKERS = NUM_CORES * NUM_SUBCORES
BAGS_PER_WORKER = BATCH // NUM_WORKERS
CHUNK = 16
NUM_CHUNKS = BAGS_PER_WORKER // CHUNK
ROWS = CHUNK * HIST
NLANE = 16
DCHUNKS = EMBED_DIM // NLANE

_mesh = plsc.VectorSubcoreMesh(
    core_axis_name="c", subcore_axis_name="s",
    num_cores=NUM_CORES, num_subcores=NUM_SUBCORES)


@functools.partial(
    pl.kernel,
    mesh=_mesh,
    out_type=jax.ShapeDtypeStruct((BATCH, EMBED_DIM), jnp.float32),
    scratch_types=[
        pltpu.VMEM((2, ROWS), jnp.int32),
        pltpu.VMEM((ROWS, EMBED_DIM), jnp.float32),
        pltpu.VMEM((ROWS, EMBED_DIM), jnp.float32),
        pltpu.VMEM((CHUNK, EMBED_DIM), jnp.float32),
        pltpu.VMEM((CHUNK, EMBED_DIM), jnp.float32),
        pltpu.SemaphoreType.DMA,
        pltpu.SemaphoreType.DMA,
        pltpu.SemaphoreType.DMA,
        pltpu.SemaphoreType.DMA,
        pltpu.SemaphoreType.DMA,
        pltpu.SemaphoreType.DMA,
    ],
    compiler_params=pltpu.CompilerParams(use_tc_tiling_on_sc=False),
)
def _embed_bag_mean(idx_hbm, w_hbm, out_hbm, idx_v, rows0_v, rows1_v,
                    out0_v, out1_v, gsem0, gsem1, osem0, osem1, isem0, isem1):
    wid = lax.axis_index("s") * NUM_CORES + lax.axis_index("c")
    base_bag = wid * BAGS_PER_WORKER
    rows_bufs = (rows0_v, rows1_v)
    out_bufs = (out0_v, out1_v)
    gsems = (gsem0, gsem1)
    osems = (osem0, osem1)
    isems = (isem0, isem1)

    def idx_slice(ci):
        return idx_hbm.at[pl.ds((base_bag + ci * CHUNK) * HIST, ROWS)]

    pltpu.sync_copy(idx_slice(0), idx_v.at[0])
    pltpu.async_copy(w_hbm.at[idx_v.at[0]], rows_bufs[0], gsems[0])
    pltpu.async_copy(idx_slice(1), idx_v.at[1], isems[1])

    def process(rows_v, out_v):
        def bag_body(bi, carry2):
            acc = [rows_v[bi * HIST, pl.ds(c * NLANE, NLANE)]
                   for c in range(DCHUNKS)]
            for ri in range(1, HIST):
                for c in range(DCHUNKS):
                    acc[c] = acc[c] + rows_v[bi * HIST + ri,
                                             pl.ds(c * NLANE, NLANE)]
            for c in range(DCHUNKS):
                out_v[bi, pl.ds(c * NLANE, NLANE)] = jnp.maximum(
                    acc[c] * (1.0 / HIST), 0.0)
            return carry2

        lax.fori_loop(0, CHUNK, bag_body, 0)

    def outer(ci2, carry):
        for b in range(2):
            ci = ci2 * 2 + b
            nb = 1 - b

            @pl.when(ci + 1 < NUM_CHUNKS)
            def _():
                pltpu.make_async_copy(idx_slice(ci + 1), idx_v.at[nb],
                                      isems[nb]).wait()
                pltpu.async_copy(w_hbm.at[idx_v.at[nb]], rows_bufs[nb],
                                 gsems[nb])

            pltpu.make_async_copy(w_hbm.at[idx_v.at[b]],
                                  rows_bufs[b], gsems[b]).wait()

            @pl.when(ci + 2 < NUM_CHUNKS)
            def _():
                pltpu.async_copy(idx_slice(ci + 2), idx_v.at[b], isems[b])

            @pl.when(ci >= 2)
            def _():
                pltpu.make_async_copy(
                    out_bufs[b], out_hbm.at[pl.ds(base_bag, CHUNK)],
                    osems[b]).wait()

            process(rows_bufs[b], out_bufs[b])
            pltpu.async_copy(out_bufs[b],
                             out_hbm.at[pl.ds(base_bag + ci * CHUNK, CHUNK)],
                             osems[b])
        return carry

    lax.fori_loop(0, NUM_CHUNKS // 2, outer, 0)

    for b in range(2):
        pltpu.make_async_copy(out_bufs[b],
                              out_hbm.at[pl.ds(base_bag, CHUNK)],
                              osems[b]).wait()


def kernel(indices, W):
    idx_flat = indices.reshape(-1).astype(jnp.int32)
    return _embed_bag_mean(idx_flat, W)


# speedup vs baseline: 23.4181x; 1.0443x over previous
import functools

import jax
import jax.numpy as jnp
from jax import lax
from jax.experimental import pallas as pl
from jax.experimental.pallas import tpu as pltpu
from jax.experimental.pallas import tpu_sc as plsc

VOCAB = 100000
EMBED_DIM = 64
BATCH = 16384
HIST = 50

NUM_CORES = 2
NUM_SUBCORES = 16
NUM_WORKERS = NUM_CORES * NUM_SUBCORES
BAGS_PER_WORKER = BATCH // NUM_WORKERS
CHUNK = 16
NUM_CHUNKS = BAGS_PER_WORKER // CHUNK
ROWS = CHUNK * HIST
NLANE = 16
DCHUNKS = EMBED_DIM // NLANE

_mesh = plsc.VectorSubcoreMesh(
    core_axis_name="c", subcore_axis_name="s",
    num_cores=NUM_CORES, num_subcores=NUM_SUBCORES)


@functools.partial(
    pl.kernel,
    mesh=_mesh,
    out_type=jax.ShapeDtypeStruct((BATCH * EMBED_DIM,), jnp.float32),
    scratch_types=[
        pltpu.VMEM((2, ROWS), jnp.int32),
        pltpu.VMEM((ROWS, EMBED_DIM), jnp.float32),
        pltpu.VMEM((ROWS, EMBED_DIM), jnp.float32),
        pltpu.VMEM((CHUNK * EMBED_DIM,), jnp.float32),
        pltpu.VMEM((CHUNK * EMBED_DIM,), jnp.float32),
        pltpu.SemaphoreType.DMA,
        pltpu.SemaphoreType.DMA,
        pltpu.SemaphoreType.DMA,
        pltpu.SemaphoreType.DMA,
        pltpu.SemaphoreType.DMA,
        pltpu.SemaphoreType.DMA,
    ],
    compiler_params=pltpu.CompilerParams(use_tc_tiling_on_sc=False),
)
def _embed_bag_mean(idx_hbm, w_hbm, out_hbm, idx_v, rows0_v, rows1_v,
                    out0_v, out1_v, gsem0, gsem1, osem0, osem1, isem0, isem1):
    wid = lax.axis_index("s") * NUM_CORES + lax.axis_index("c")
    base_bag = wid * BAGS_PER_WORKER
    rows_bufs = (rows0_v, rows1_v)
    out_bufs = (out0_v, out1_v)
    gsems = (gsem0, gsem1)
    osems = (osem0, osem1)
    isems = (isem0, isem1)

    def idx_slice(ci):
        return idx_hbm.at[pl.ds((base_bag + ci * CHUNK) * HIST, ROWS)]

    pltpu.sync_copy(idx_slice(0), idx_v.at[0])
    pltpu.async_copy(w_hbm.at[idx_v.at[0]], rows_bufs[0], gsems[0])
    pltpu.async_copy(idx_slice(1), idx_v.at[1], isems[1])

    def process(rows_v, out_v):
        def bag_body(bi, carry2):
            acc = [rows_v[bi * HIST, pl.ds(c * NLANE, NLANE)]
                   for c in range(DCHUNKS)]
            for ri in range(1, HIST):
                for c in range(DCHUNKS):
                    acc[c] = acc[c] + rows_v[bi * HIST + ri,
                                             pl.ds(c * NLANE, NLANE)]
            for c in range(DCHUNKS):
                out_v[pl.ds(bi * EMBED_DIM + c * NLANE, NLANE)] = jnp.maximum(
                    acc[c] * (1.0 / HIST), 0.0)
            return carry2

        lax.fori_loop(0, CHUNK, bag_body, 0)

    def outer(ci2, carry):
        for b in range(2):
            ci = ci2 * 2 + b
            nb = 1 - b

            @pl.when(ci + 1 < NUM_CHUNKS)
            def _():
                pltpu.make_async_copy(idx_slice(ci + 1), idx_v.at[nb],
                                      isems[nb]).wait()
                pltpu.async_copy(w_hbm.at[idx_v.at[nb]], rows_bufs[nb],
                                 gsems[nb])

            pltpu.make_async_copy(w_hbm.at[idx_v.at[b]],
                                  rows_bufs[b], gsems[b]).wait()

            @pl.when(ci + 2 < NUM_CHUNKS)
            def _():
                pltpu.async_copy(idx_slice(ci + 2), idx_v.at[b], isems[b])

            @pl.when(ci >= 2)
            def _():
                pltpu.make_async_copy(
                    out_bufs[b],
                    out_hbm.at[pl.ds(base_bag * EMBED_DIM, CHUNK * EMBED_DIM)],
                    osems[b]).wait()

            process(rows_bufs[b], out_bufs[b])
            pltpu.async_copy(
                out_bufs[b],
                out_hbm.at[pl.ds((base_bag + ci * CHUNK) * EMBED_DIM,
                                 CHUNK * EMBED_DIM)],
                osems[b])
        return carry

    lax.fori_loop(0, NUM_CHUNKS // 2, outer, 0)

    for b in range(2):
        pltpu.make_async_copy(
            out_bufs[b],
            out_hbm.at[pl.ds(base_bag * EMBED_DIM, CHUNK * EMBED_DIM)],
            osems[b]).wait()


def kernel(indices, W):
    idx2 = (indices.astype(jnp.int32) * 2).reshape(-1)
    w_pad = jnp.pad(W, ((0, 0), (0, EMBED_DIM))).reshape(2 * VOCAB, EMBED_DIM)
    out_flat = _embed_bag_mean(idx2, w_pad)
    return out_flat.reshape(BATCH, EMBED_DIM)
